# baseline (speedup 1.0000x reference)
import functools

import jax
import jax.numpy as jnp
from jax import lax
from jax.experimental import pallas as pl
from jax.experimental.pallas import tpu as pltpu

_EPS = 1e-5
_VMEM_LIMIT = 64 * 1024 * 1024


def _conv_bn_flat_kernel(x_ref, w_ref, g_ref, b_ref, *rest, nb, lout, lbuf,
                         relu_in, relu_out, has_skip, emit_raw, eps):
    idx = 0
    skip_ref = None
    if has_skip:
        skip_ref = rest[idx]
        idx += 1
    o_ref = rest[idx]
    idx += 1
    raw_ref = rest[idx] if emit_raw else None

    xf = x_ref[0]
    if relu_in:
        xf = jnp.maximum(xf, 0.0)
    w = w_ref[0]
    kp = w.shape[0]
    cout = w.shape[1]

    span = nb * lbuf - kp + 1
    acc = jnp.zeros((cout, span), jnp.float32)
    for k in range(kp):
        acc = acc + jnp.dot(w[k], xf[:, k:k + span],
                            preferred_element_type=jnp.float32)

    ssum = jnp.zeros((cout, 1), jnp.float32)
    ssq = jnp.zeros((cout, 1), jnp.float32)
    ys = []
    for n in range(nb):
        yn = acc[:, n * lbuf:n * lbuf + lout]
        if has_skip:
            yn = yn + skip_ref[0][:, n * lout:(n + 1) * lout]
        if emit_raw:
            raw_ref[0, :, n * lout:(n + 1) * lout] = yn
        ssum = ssum + jnp.sum(yn, axis=1, keepdims=True)
        ssq = ssq + jnp.sum(yn * yn, axis=1, keepdims=True)
        ys.append(yn)

    inv_m = 1.0 / float(nb * lout)
    mean = ssum * inv_m
    var = ssq * inv_m - mean * mean
    scale = g_ref[0] * lax.rsqrt(var + eps)
    shift = b_ref[0] - mean * scale
    for n in range(nb):
        y = ys[n] * scale + shift
        if relu_out:
            y = jnp.maximum(y, 0.0)
        o_ref[0, :, n * lout:(n + 1) * lout] = y


def _linear_bn_flat_kernel(x_ref, w_ref, bias_ref, g_ref, b_ref, o_ref,
                           *, nb, nch, relu_in, relu_out, eps):
    x = x_ref[0]
    if relu_in:
        x = jnp.maximum(x, 0.0)
    y = jnp.dot(x, w_ref[0], preferred_element_type=jnp.float32) + bias_ref[0]

    nrows = nb * nch
    lout = y.shape[1]
    rs = jnp.sum(y, axis=1, keepdims=True)
    rq = jnp.sum(y * y, axis=1, keepdims=True)
    row_ch = lax.broadcasted_iota(jnp.int32, (nch, nrows), 1) % nch
    ch_id = lax.broadcasted_iota(jnp.int32, (nch, nrows), 0)
    m = (row_ch == ch_id).astype(jnp.float32)

    inv_m = 1.0 / float(nb * lout)
    mean = jnp.dot(m, rs, preferred_element_type=jnp.float32) * inv_m
    ex2 = jnp.dot(m, rq, preferred_element_type=jnp.float32) * inv_m
    var = ex2 - mean * mean
    scale = g_ref[0] * lax.rsqrt(var + eps)
    shift = b_ref[0] - mean * scale
    sc_rows = jnp.dot(m.T, scale, preferred_element_type=jnp.float32)
    sh_rows = jnp.dot(m.T, shift, preferred_element_type=jnp.float32)

    y = y * sc_rows + sh_rows
    if relu_out:
        y = jnp.maximum(y, 0.0)
    o_ref[0] = y


def _fold_flat(x, w, stride, plo, phi):
    n, _, length = x.shape
    cout, _, k = w.shape
    xp = jnp.pad(x, ((0, 0), (0, 0), (plo, phi)))
    lpad = length + plo + phi
    lout = (lpad - k) // stride + 1
    if stride == 1:
        kp = k
        xb, wf = xp, w
    else:
        kp = -(-k // stride)
        lbuf = lout + kp - 1
        xs, ws = [], []
        for p in range(stride):
            xq = xp[:, :, p::stride][:, :, :lbuf]
            if xq.shape[2] < lbuf:
                xq = jnp.pad(xq, ((0, 0), (0, 0), (0, lbuf - xq.shape[2])))
            xs.append(xq)
            wq = w[:, :, p::stride]
            if wq.shape[2] < kp:
                wq = jnp.pad(wq, ((0, 0), (0, 0), (0, kp - wq.shape[2])))
            ws.append(wq)
        xb = jnp.concatenate(xs, axis=1)
        wf = jnp.concatenate(ws, axis=1)
    lbuf = lout + kp - 1
    cg = xb.shape[1]
    if cg % 8:
        cpad = 8 - cg % 8
        xb = jnp.pad(xb, ((0, 0), (0, cpad), (0, 0)))
        wf = jnp.pad(wf, ((0, 0), (0, cpad), (0, 0)))
        cg += cpad
    xf = jnp.transpose(xb, (1, 0, 2)).reshape(cg, n * lbuf)
    wb = jnp.transpose(wf, (2, 0, 1)).astype(jnp.float32)
    return xf.astype(jnp.float32), wb, lout, lbuf


def _dilate_flip(x, w_t, stride, padding, output_padding):
    n, cin, length = x.shape
    k = w_t.shape[2]
    if stride > 1:
        xz = jnp.concatenate(
            [x[..., None], jnp.zeros((n, cin, length, stride - 1), x.dtype)],
            axis=-1)
        x = xz.reshape(n, cin, length * stride)[:, :, :(length - 1) * stride + 1]
    plo = k - 1 - padding
    phi = k - 1 - padding + output_padding
    wc = jnp.transpose(w_t, (1, 0, 2))[:, :, ::-1]
    return x, wc, plo, phi


def _conv_bn(xs, ws, gs, bs, *, stride=1, pad=(0, 0), relu_in=False,
             relu_out=False, skip=None, emit_raw=False):
    gsz = len(xs)
    n = xs[0].shape[0]
    xfs, wbs = [], []
    for x, w in zip(xs, ws):
        xf, wb, lout, lbuf = _fold_flat(x, w, stride, pad[0], pad[1])
        xfs.append(xf)
        wbs.append(wb)
    cout = wbs[0].shape[1]
    X = jnp.stack(xfs)
    W = jnp.stack(wbs)
    G2 = jnp.stack([g.reshape(cout, 1).astype(jnp.float32) for g in gs])
    B2 = jnp.stack([b.reshape(cout, 1).astype(jnp.float32) for b in bs])

    inputs = [X, W, G2, B2]
    in_specs = [
        pl.BlockSpec((1,) + X.shape[1:], lambda i: (i, 0, 0)),
        pl.BlockSpec((1,) + W.shape[1:], lambda i: (i, 0, 0, 0)),
        pl.BlockSpec((1, cout, 1), lambda i: (i, 0, 0)),
        pl.BlockSpec((1, cout, 1), lambda i: (i, 0, 0)),
    ]
    if skip is not None:
        sf = jnp.transpose(skip, (1, 0, 2)).reshape(1, cout, n * lout)
        inputs.append(sf.astype(jnp.float32))
        in_specs.append(pl.BlockSpec((1, cout, n * lout), lambda i: (0, 0, 0)))

    out_sds = jax.ShapeDtypeStruct((gsz, cout, n * lout), jnp.float32)
    out_spec = pl.BlockSpec((1, cout, n * lout), lambda i: (i, 0, 0))
    if emit_raw:
        out_shape = (out_sds, out_sds)
        out_specs = (out_spec, out_spec)
    else:
        out_shape = out_sds
        out_specs = out_spec

    kern = functools.partial(
        _conv_bn_flat_kernel, nb=n, lout=lout, lbuf=lbuf, relu_in=relu_in,
        relu_out=relu_out, has_skip=skip is not None, emit_raw=emit_raw,
        eps=_EPS)
    res = pl.pallas_call(
        kern,
        out_shape=out_shape,
        grid=(gsz,),
        in_specs=in_specs,
        out_specs=out_specs,
        compiler_params=pltpu.CompilerParams(
            dimension_semantics=("parallel",),
            vmem_limit_bytes=_VMEM_LIMIT),
    )(*inputs)

    def unflat(a):
        return [jnp.transpose(a[g].reshape(cout, n, lout), (1, 0, 2))
                for g in range(gsz)]

    if emit_raw:
        return unflat(res[0]), unflat(res[1])
    return unflat(res)


def _linear_bn(xs, ws, biases, gs, bs, *, relu_in=False, relu_out=False):
    gsz = len(xs)
    n, c, lin = xs[0].shape
    lout = ws[0].shape[0]
    X = jnp.stack([x.reshape(n * c, lin).astype(jnp.float32) for x in xs])
    W = jnp.stack([jnp.transpose(w).astype(jnp.float32) for w in ws])
    BIAS = jnp.stack([b.reshape(1, lout).astype(jnp.float32) for b in biases])
    G2 = jnp.stack([g.reshape(c, 1).astype(jnp.float32) for g in gs])
    B2 = jnp.stack([b.reshape(c, 1).astype(jnp.float32) for b in bs])
    kern = functools.partial(_linear_bn_flat_kernel, nb=n, nch=c,
                             relu_in=relu_in, relu_out=relu_out, eps=_EPS)
    res = pl.pallas_call(
        kern,
        out_shape=jax.ShapeDtypeStruct((gsz, n * c, lout), jnp.float32),
        grid=(gsz,),
        in_specs=[
            pl.BlockSpec((1, n * c, lin), lambda i: (i, 0, 0)),
            pl.BlockSpec((1, lin, lout), lambda i: (i, 0, 0)),
            pl.BlockSpec((1, 1, lout), lambda i: (i, 0, 0)),
            pl.BlockSpec((1, c, 1), lambda i: (i, 0, 0)),
            pl.BlockSpec((1, c, 1), lambda i: (i, 0, 0)),
        ],
        out_specs=pl.BlockSpec((1, n * c, lout), lambda i: (i, 0, 0)),
        compiler_params=pltpu.CompilerParams(
            dimension_semantics=("parallel",),
            vmem_limit_bytes=_VMEM_LIMIT),
    )(X, W, BIAS, G2, B2)
    return [res[g].reshape(n, c, lout) for g in range(gsz)]


def _convt_bn(xs, wts, gs, bs, *, stride, padding, output_padding,
              relu_in=False, skip=None):
    xds, wcs = [], []
    plo = phi = 0
    for x, wt in zip(xs, wts):
        xd, wc, plo, phi = _dilate_flip(x, wt, stride, padding, output_padding)
        xds.append(xd)
        wcs.append(wc)
    return _conv_bn(xds, wcs, gs, bs, stride=1, pad=(plo, phi),
                    relu_in=relu_in, skip=skip)


def kernel(x, w_enc1, b_enc1, bn1_g, bn1_b, w_enc3, b_enc3, bn3_g, bn3_b,
           w_enc5, b_enc5, bn5_g, bn5_b, W_fc6, b_fc6, bn6_g, bn6_b,
           b1_W_fc, b1_b_fc, b1_bn6x_g, b1_bn6x_b,
           b1_w_dec5, b1_b_dec5, b1_bn5x_g, b1_bn5x_b,
           b1_w_dec3, b1_b_dec3, b1_bn3x_g, b1_bn3x_b,
           b1_w_dec1, b1_b_dec1, b1_bn1x_g, b1_bn1x_b,
           b2_W_fc, b2_b_fc, b2_bn6x_g, b2_bn6x_b,
           b2_w_dec5, b2_b_dec5, b2_bn5x_g, b2_bn5x_b,
           b2_w_dec3, b2_b_dec3, b2_bn3x_g, b2_bn3x_b,
           b2_w_dec1, b2_b_dec1, b2_bn1x_g, b2_bn1x_b):
    (h,) = _conv_bn([x], [w_enc1], [bn1_g], [bn1_b],
                    stride=1, pad=(22, 22), relu_out=True)
    (hs, x0s) = _conv_bn([h], [w_enc3], [bn3_g], [bn3_b],
                         stride=2, pad=(10, 10), relu_out=True, emit_raw=True)
    h, x0 = hs[0], x0s[0]
    (h,) = _conv_bn([h], [w_enc5], [bn5_g], [bn5_b],
                    stride=1, pad=(4, 4), relu_out=True)
    (h,) = _linear_bn([h], [W_fc6], [b_fc6], [bn6_g], [bn6_b], relu_out=True)

    z = _linear_bn([h, h], [b1_W_fc, b2_W_fc], [b1_b_fc, b2_b_fc],
                   [b1_bn6x_g, b2_bn6x_g], [b1_bn6x_b, b2_bn6x_b],
                   relu_in=True)
    u = _convt_bn(z, [b1_w_dec5, b2_w_dec5], [b1_bn5x_g, b2_bn5x_g],
                  [b1_bn5x_b, b2_bn5x_b], stride=1, padding=4,
                  output_padding=0, relu_in=True, skip=x0)
    v = _convt_bn(u, [b1_w_dec3, b2_w_dec3], [b1_bn3x_g, b2_bn3x_g],
                  [b1_bn3x_b, b2_bn3x_b], stride=2, padding=10,
                  output_padding=1, relu_in=True)
    o = _convt_bn(v, [b1_w_dec1, b2_w_dec1], [b1_bn1x_g, b2_bn1x_g],
                  [b1_bn1x_b, b2_bn1x_b], stride=1, padding=22,
                  output_padding=0, relu_in=True)
    return (o[0], o[1])


# speedup vs baseline: 1.0782x; 1.0782x over previous
import functools

import jax
import jax.numpy as jnp
from jax import lax
from jax.experimental import pallas as pl
from jax.experimental.pallas import tpu as pltpu

_EPS = 1e-5
_VMEM_LIMIT = 64 * 1024 * 1024


def _conv_bn_flat_kernel(x_ref, w_ref, g_ref, b_ref, *rest, nb, lout, lbuf,
                         relu_in, relu_out, has_skip, emit_raw, eps):
    idx = 0
    skip_ref = None
    if has_skip:
        skip_ref = rest[idx]
        idx += 1
    o_ref = rest[idx]
    idx += 1
    raw_ref = rest[idx] if emit_raw else None

    xf = jnp.concatenate([x_ref[0, n] for n in range(nb)], axis=1)
    if relu_in:
        xf = jnp.maximum(xf, 0.0)
    w = w_ref[0]
    kp = w.shape[0]
    cout = w.shape[1]

    span = nb * lbuf - kp + 1
    acc = jnp.zeros((cout, span), jnp.float32)
    for k in range(kp):
        acc = acc + jnp.dot(w[k], xf[:, k:k + span],
                            preferred_element_type=jnp.float32)

    ssum = jnp.zeros((cout, 1), jnp.float32)
    ssq = jnp.zeros((cout, 1), jnp.float32)
    ys = []
    for n in range(nb):
        yn = acc[:, n * lbuf:n * lbuf + lout]
        if has_skip:
            yn = yn + skip_ref[0, n]
        if emit_raw:
            raw_ref[0, n] = yn
        ssum = ssum + jnp.sum(yn, axis=1, keepdims=True)
        ssq = ssq + jnp.sum(yn * yn, axis=1, keepdims=True)
        ys.append(yn)

    inv_m = 1.0 / float(nb * lout)
    mean = ssum * inv_m
    var = ssq * inv_m - mean * mean
    scale = g_ref[0] * lax.rsqrt(var + eps)
    shift = b_ref[0] - mean * scale
    for n in range(nb):
        y = ys[n] * scale + shift
        if relu_out:
            y = jnp.maximum(y, 0.0)
        o_ref[0, n] = y


def _linear_bn_flat_kernel(x_ref, w_ref, bias_ref, g_ref, b_ref, o_ref,
                           *, nb, nch, relu_in, relu_out, eps):
    x = x_ref[0]
    if relu_in:
        x = jnp.maximum(x, 0.0)
    y = jnp.dot(x, w_ref[0], preferred_element_type=jnp.float32) + bias_ref[0]

    nrows = nb * nch
    lout = y.shape[1]
    rs = jnp.sum(y, axis=1, keepdims=True)
    rq = jnp.sum(y * y, axis=1, keepdims=True)
    row_ch = lax.broadcasted_iota(jnp.int32, (nch, nrows), 1) % nch
    ch_id = lax.broadcasted_iota(jnp.int32, (nch, nrows), 0)
    m = (row_ch == ch_id).astype(jnp.float32)

    inv_m = 1.0 / float(nb * lout)
    mean = jnp.dot(m, rs, preferred_element_type=jnp.float32) * inv_m
    ex2 = jnp.dot(m, rq, preferred_element_type=jnp.float32) * inv_m
    var = ex2 - mean * mean
    scale = g_ref[0] * lax.rsqrt(var + eps)
    shift = b_ref[0] - mean * scale
    sc_rows = jnp.dot(m.T, scale, preferred_element_type=jnp.float32)
    sh_rows = jnp.dot(m.T, shift, preferred_element_type=jnp.float32)

    y = y * sc_rows + sh_rows
    if relu_out:
        y = jnp.maximum(y, 0.0)
    o_ref[0] = y


def _fold_flat(x, w, stride, plo, phi):
    n, _, length = x.shape
    cout, _, k = w.shape
    xp = jnp.pad(x, ((0, 0), (0, 0), (plo, phi)))
    lpad = length + plo + phi
    lout = (lpad - k) // stride + 1
    if stride == 1:
        kp = k
        xb, wf = xp, w
    else:
        kp = -(-k // stride)
        lbuf = lout + kp - 1
        xs, ws = [], []
        for p in range(stride):
            xq = xp[:, :, p::stride][:, :, :lbuf]
            if xq.shape[2] < lbuf:
                xq = jnp.pad(xq, ((0, 0), (0, 0), (0, lbuf - xq.shape[2])))
            xs.append(xq)
            wq = w[:, :, p::stride]
            if wq.shape[2] < kp:
                wq = jnp.pad(wq, ((0, 0), (0, 0), (0, kp - wq.shape[2])))
            ws.append(wq)
        xb = jnp.concatenate(xs, axis=1)
        wf = jnp.concatenate(ws, axis=1)
    lbuf = lout + kp - 1
    cg = xb.shape[1]
    if cg % 8:
        cpad = 8 - cg % 8
        xb = jnp.pad(xb, ((0, 0), (0, cpad), (0, 0)))
        wf = jnp.pad(wf, ((0, 0), (0, cpad), (0, 0)))
        cg += cpad
    wb = jnp.transpose(wf, (2, 0, 1)).astype(jnp.float32)
    return xb.astype(jnp.float32), wb, lout, lbuf


def _dilate_flip(x, w_t, stride, padding, output_padding):
    n, cin, length = x.shape
    k = w_t.shape[2]
    if stride > 1:
        xz = jnp.concatenate(
            [x[..., None], jnp.zeros((n, cin, length, stride - 1), x.dtype)],
            axis=-1)
        x = xz.reshape(n, cin, length * stride)[:, :, :(length - 1) * stride + 1]
    plo = k - 1 - padding
    phi = k - 1 - padding + output_padding
    wc = jnp.transpose(w_t, (1, 0, 2))[:, :, ::-1]
    return x, wc, plo, phi


def _conv_bn(xs, ws, gs, bs, *, stride=1, pad=(0, 0), relu_in=False,
             relu_out=False, skip=None, emit_raw=False):
    gsz = len(xs)
    n = xs[0].shape[0]
    xfs, wbs = [], []
    for x, w in zip(xs, ws):
        xf, wb, lout, lbuf = _fold_flat(x, w, stride, pad[0], pad[1])
        xfs.append(xf)
        wbs.append(wb)
    cout = wbs[0].shape[1]
    X = jnp.stack(xfs)
    W = jnp.stack(wbs)
    G2 = jnp.stack([g.reshape(cout, 1).astype(jnp.float32) for g in gs])
    B2 = jnp.stack([b.reshape(cout, 1).astype(jnp.float32) for b in bs])

    inputs = [X, W, G2, B2]
    in_specs = [
        pl.BlockSpec((1,) + X.shape[1:], lambda i: (i, 0, 0, 0)),
        pl.BlockSpec((1,) + W.shape[1:], lambda i: (i, 0, 0, 0)),
        pl.BlockSpec((1, cout, 1), lambda i: (i, 0, 0)),
        pl.BlockSpec((1, cout, 1), lambda i: (i, 0, 0)),
    ]
    if skip is not None:
        sf = skip[None].astype(jnp.float32)
        inputs.append(sf)
        in_specs.append(
            pl.BlockSpec((1, n, cout, lout), lambda i: (0, 0, 0, 0)))

    out_sds = jax.ShapeDtypeStruct((gsz, n, cout, lout), jnp.float32)
    out_spec = pl.BlockSpec((1, n, cout, lout), lambda i: (i, 0, 0, 0))
    if emit_raw:
        out_shape = (out_sds, out_sds)
        out_specs = (out_spec, out_spec)
    else:
        out_shape = out_sds
        out_specs = out_spec

    kern = functools.partial(
        _conv_bn_flat_kernel, nb=n, lout=lout, lbuf=lbuf, relu_in=relu_in,
        relu_out=relu_out, has_skip=skip is not None, emit_raw=emit_raw,
        eps=_EPS)
    res = pl.pallas_call(
        kern,
        out_shape=out_shape,
        grid=(gsz,),
        in_specs=in_specs,
        out_specs=out_specs,
        compiler_params=pltpu.CompilerParams(
            dimension_semantics=("parallel",),
            vmem_limit_bytes=_VMEM_LIMIT),
    )(*inputs)

    def split(a):
        return [a[g] for g in range(gsz)]

    if emit_raw:
        return split(res[0]), split(res[1])
    return split(res)


def _linear_bn(xs, ws, biases, gs, bs, *, relu_in=False, relu_out=False):
    gsz = len(xs)
    n, c, lin = xs[0].shape
    lout = ws[0].shape[0]
    X = jnp.stack([x.reshape(n * c, lin).astype(jnp.float32) for x in xs])
    W = jnp.stack([jnp.transpose(w).astype(jnp.float32) for w in ws])
    BIAS = jnp.stack([b.reshape(1, lout).astype(jnp.float32) for b in biases])
    G2 = jnp.stack([g.reshape(c, 1).astype(jnp.float32) for g in gs])
    B2 = jnp.stack([b.reshape(c, 1).astype(jnp.float32) for b in bs])
    kern = functools.partial(_linear_bn_flat_kernel, nb=n, nch=c,
                             relu_in=relu_in, relu_out=relu_out, eps=_EPS)
    res = pl.pallas_call(
        kern,
        out_shape=jax.ShapeDtypeStruct((gsz, n * c, lout), jnp.float32),
        grid=(gsz,),
        in_specs=[
            pl.BlockSpec((1, n * c, lin), lambda i: (i, 0, 0)),
            pl.BlockSpec((1, lin, lout), lambda i: (i, 0, 0)),
            pl.BlockSpec((1, 1, lout), lambda i: (i, 0, 0)),
            pl.BlockSpec((1, c, 1), lambda i: (i, 0, 0)),
            pl.BlockSpec((1, c, 1), lambda i: (i, 0, 0)),
        ],
        out_specs=pl.BlockSpec((1, n * c, lout), lambda i: (i, 0, 0)),
        compiler_params=pltpu.CompilerParams(
            dimension_semantics=("parallel",),
            vmem_limit_bytes=_VMEM_LIMIT),
    )(X, W, BIAS, G2, B2)
    return [res[g].reshape(n, c, lout) for g in range(gsz)]


def _convt_bn(xs, wts, gs, bs, *, stride, padding, output_padding,
              relu_in=False, skip=None):
    xds, wcs = [], []
    plo = phi = 0
    for x, wt in zip(xs, wts):
        xd, wc, plo, phi = _dilate_flip(x, wt, stride, padding, output_padding)
        xds.append(xd)
        wcs.append(wc)
    return _conv_bn(xds, wcs, gs, bs, stride=1, pad=(plo, phi),
                    relu_in=relu_in, skip=skip)


def kernel(x, w_enc1, b_enc1, bn1_g, bn1_b, w_enc3, b_enc3, bn3_g, bn3_b,
           w_enc5, b_enc5, bn5_g, bn5_b, W_fc6, b_fc6, bn6_g, bn6_b,
           b1_W_fc, b1_b_fc, b1_bn6x_g, b1_bn6x_b,
           b1_w_dec5, b1_b_dec5, b1_bn5x_g, b1_bn5x_b,
           b1_w_dec3, b1_b_dec3, b1_bn3x_g, b1_bn3x_b,
           b1_w_dec1, b1_b_dec1, b1_bn1x_g, b1_bn1x_b,
           b2_W_fc, b2_b_fc, b2_bn6x_g, b2_bn6x_b,
           b2_w_dec5, b2_b_dec5, b2_bn5x_g, b2_bn5x_b,
           b2_w_dec3, b2_b_dec3, b2_bn3x_g, b2_bn3x_b,
           b2_w_dec1, b2_b_dec1, b2_bn1x_g, b2_bn1x_b):
    (h,) = _conv_bn([x], [w_enc1], [bn1_g], [bn1_b],
                    stride=1, pad=(22, 22), relu_out=True)
    (hs, x0s) = _conv_bn([h], [w_enc3], [bn3_g], [bn3_b],
                         stride=2, pad=(10, 10), relu_out=True, emit_raw=True)
    h, x0 = hs[0], x0s[0]
    (h,) = _conv_bn([h], [w_enc5], [bn5_g], [bn5_b],
                    stride=1, pad=(4, 4), relu_out=True)
    (h,) = _linear_bn([h], [W_fc6], [b_fc6], [bn6_g], [bn6_b], relu_out=True)

    z = _linear_bn([h, h], [b1_W_fc, b2_W_fc], [b1_b_fc, b2_b_fc],
                   [b1_bn6x_g, b2_bn6x_g], [b1_bn6x_b, b2_bn6x_b],
                   relu_in=True)
    u = _convt_bn(z, [b1_w_dec5, b2_w_dec5], [b1_bn5x_g, b2_bn5x_g],
                  [b1_bn5x_b, b2_bn5x_b], stride=1, padding=4,
                  output_padding=0, relu_in=True, skip=x0)
    v = _convt_bn(u, [b1_w_dec3, b2_w_dec3], [b1_bn3x_g, b2_bn3x_g],
                  [b1_bn3x_b, b2_bn3x_b], stride=2, padding=10,
                  output_padding=1, relu_in=True)
    o = _convt_bn(v, [b1_w_dec1, b2_w_dec1], [b1_bn1x_g, b2_bn1x_g],
                  [b1_bn1x_b, b2_bn1x_b], stride=1, padding=22,
                  output_padding=0, relu_in=True)
    return (o[0], o[1])


# speedup vs baseline: 1.0958x; 1.0164x over previous
import functools

import jax
import jax.numpy as jnp
from jax import lax
from jax.experimental import pallas as pl
from jax.experimental.pallas import tpu as pltpu

_EPS = 1e-5
_VMEM_LIMIT = 64 * 1024 * 1024


def _conv_bn_flat_kernel(x_ref, w_ref, g_ref, b_ref, *rest, nb, lout, lbuf,
                         relu_in, relu_out, has_skip, emit_raw, eps):
    idx = 0
    skip_ref = None
    if has_skip:
        skip_ref = rest[idx]
        idx += 1
    o_ref = rest[idx]
    idx += 1
    raw_ref = rest[idx] if emit_raw else None

    w = w_ref[0]
    kp = w.shape[0]
    cout = w.shape[1]

    ssum = jnp.zeros((cout, 1), jnp.float32)
    ssq = jnp.zeros((cout, 1), jnp.float32)
    ys = []
    for n in range(nb):
        xn = x_ref[0, n]
        if relu_in:
            xn = jnp.maximum(xn, 0.0)
        yn = jnp.zeros((cout, lout), jnp.float32)
        for k in range(kp):
            yn = yn + jnp.dot(w[k], xn[:, k:k + lout],
                              preferred_element_type=jnp.float32)
        if has_skip:
            yn = yn + skip_ref[0, n]
        if emit_raw:
            raw_ref[0, n] = yn
        ssum = ssum + jnp.sum(yn, axis=1, keepdims=True)
        ssq = ssq + jnp.sum(yn * yn, axis=1, keepdims=True)
        ys.append(yn)

    inv_m = 1.0 / float(nb * lout)
    mean = ssum * inv_m
    var = ssq * inv_m - mean * mean
    scale = g_ref[0] * lax.rsqrt(var + eps)
    shift = b_ref[0] - mean * scale
    for n in range(nb):
        y = ys[n] * scale + shift
        if relu_out:
            y = jnp.maximum(y, 0.0)
        o_ref[0, n] = y


def _linear_bn_flat_kernel(x_ref, w_ref, bias_ref, g_ref, b_ref, o_ref,
                           *, nb, nch, relu_in, relu_out, eps):
    x = x_ref[0]
    if relu_in:
        x = jnp.maximum(x, 0.0)
    y = jnp.dot(x, w_ref[0], preferred_element_type=jnp.float32) + bias_ref[0]

    nrows = nb * nch
    lout = y.shape[1]
    rs = jnp.sum(y, axis=1, keepdims=True)
    rq = jnp.sum(y * y, axis=1, keepdims=True)
    row_ch = lax.broadcasted_iota(jnp.int32, (nch, nrows), 1) % nch
    ch_id = lax.broadcasted_iota(jnp.int32, (nch, nrows), 0)
    m = (row_ch == ch_id).astype(jnp.float32)

    inv_m = 1.0 / float(nb * lout)
    mean = jnp.dot(m, rs, preferred_element_type=jnp.float32) * inv_m
    ex2 = jnp.dot(m, rq, preferred_element_type=jnp.float32) * inv_m
    var = ex2 - mean * mean
    scale = g_ref[0] * lax.rsqrt(var + eps)
    shift = b_ref[0] - mean * scale
    sc_rows = jnp.dot(m.T, scale, preferred_element_type=jnp.float32)
    sh_rows = jnp.dot(m.T, shift, preferred_element_type=jnp.float32)

    y = y * sc_rows + sh_rows
    if relu_out:
        y = jnp.maximum(y, 0.0)
    o_ref[0] = y


def _fold_flat(x, w, stride, plo, phi):
    n, _, length = x.shape
    cout, _, k = w.shape
    xp = jnp.pad(x, ((0, 0), (0, 0), (plo, phi)))
    lpad = length + plo + phi
    lout = (lpad - k) // stride + 1
    if stride == 1:
        kp = k
        xb, wf = xp, w
    else:
        kp = -(-k // stride)
        lbuf = lout + kp - 1
        xs, ws = [], []
        for p in range(stride):
            xq = xp[:, :, p::stride][:, :, :lbuf]
            if xq.shape[2] < lbuf:
                xq = jnp.pad(xq, ((0, 0), (0, 0), (0, lbuf - xq.shape[2])))
            xs.append(xq)
            wq = w[:, :, p::stride]
            if wq.shape[2] < kp:
                wq = jnp.pad(wq, ((0, 0), (0, 0), (0, kp - wq.shape[2])))
            ws.append(wq)
        xb = jnp.concatenate(xs, axis=1)
        wf = jnp.concatenate(ws, axis=1)
    lbuf = lout + kp - 1
    cg = xb.shape[1]
    if cg % 8:
        cpad = 8 - cg % 8
        xb = jnp.pad(xb, ((0, 0), (0, cpad), (0, 0)))
        wf = jnp.pad(wf, ((0, 0), (0, cpad), (0, 0)))
        cg += cpad
    wb = jnp.transpose(wf, (2, 0, 1)).astype(jnp.float32)
    return xb.astype(jnp.float32), wb, lout, lbuf


def _dilate_flip(x, w_t, stride, padding, output_padding):
    n, cin, length = x.shape
    k = w_t.shape[2]
    if stride > 1:
        xz = jnp.concatenate(
            [x[..., None], jnp.zeros((n, cin, length, stride - 1), x.dtype)],
            axis=-1)
        x = xz.reshape(n, cin, length * stride)[:, :, :(length - 1) * stride + 1]
    plo = k - 1 - padding
    phi = k - 1 - padding + output_padding
    wc = jnp.transpose(w_t, (1, 0, 2))[:, :, ::-1]
    return x, wc, plo, phi


def _conv_bn(xs, ws, gs, bs, *, stride=1, pad=(0, 0), relu_in=False,
             relu_out=False, skip=None, emit_raw=False):
    gsz = len(xs)
    n = xs[0].shape[0]
    xfs, wbs = [], []
    for x, w in zip(xs, ws):
        xf, wb, lout, lbuf = _fold_flat(x, w, stride, pad[0], pad[1])
        xfs.append(xf)
        wbs.append(wb)
    cout = wbs[0].shape[1]
    X = jnp.stack(xfs)
    W = jnp.stack(wbs)
    G2 = jnp.stack([g.reshape(cout, 1).astype(jnp.float32) for g in gs])
    B2 = jnp.stack([b.reshape(cout, 1).astype(jnp.float32) for b in bs])

    inputs = [X, W, G2, B2]
    in_specs = [
        pl.BlockSpec((1,) + X.shape[1:], lambda i: (i, 0, 0, 0)),
        pl.BlockSpec((1,) + W.shape[1:], lambda i: (i, 0, 0, 0)),
        pl.BlockSpec((1, cout, 1), lambda i: (i, 0, 0)),
        pl.BlockSpec((1, cout, 1), lambda i: (i, 0, 0)),
    ]
    if skip is not None:
        sf = skip[None].astype(jnp.float32)
        inputs.append(sf)
        in_specs.append(
            pl.BlockSpec((1, n, cout, lout), lambda i: (0, 0, 0, 0)))

    out_sds = jax.ShapeDtypeStruct((gsz, n, cout, lout), jnp.float32)
    out_spec = pl.BlockSpec((1, n, cout, lout), lambda i: (i, 0, 0, 0))
    if emit_raw:
        out_shape = (out_sds, out_sds)
        out_specs = (out_spec, out_spec)
    else:
        out_shape = out_sds
        out_specs = out_spec

    kern = functools.partial(
        _conv_bn_flat_kernel, nb=n, lout=lout, lbuf=lbuf, relu_in=relu_in,
        relu_out=relu_out, has_skip=skip is not None, emit_raw=emit_raw,
        eps=_EPS)
    res = pl.pallas_call(
        kern,
        out_shape=out_shape,
        grid=(gsz,),
        in_specs=in_specs,
        out_specs=out_specs,
        compiler_params=pltpu.CompilerParams(
            dimension_semantics=("parallel",),
            vmem_limit_bytes=_VMEM_LIMIT),
    )(*inputs)

    def split(a):
        return [a[g] for g in range(gsz)]

    if emit_raw:
        return split(res[0]), split(res[1])
    return split(res)


def _linear_bn(xs, ws, biases, gs, bs, *, relu_in=False, relu_out=False):
    gsz = len(xs)
    n, c, lin = xs[0].shape
    lout = ws[0].shape[0]
    X = jnp.stack([x.reshape(n * c, lin).astype(jnp.float32) for x in xs])
    W = jnp.stack([jnp.transpose(w).astype(jnp.float32) for w in ws])
    BIAS = jnp.stack([b.reshape(1, lout).astype(jnp.float32) for b in biases])
    G2 = jnp.stack([g.reshape(c, 1).astype(jnp.float32) for g in gs])
    B2 = jnp.stack([b.reshape(c, 1).astype(jnp.float32) for b in bs])
    kern = functools.partial(_linear_bn_flat_kernel, nb=n, nch=c,
                             relu_in=relu_in, relu_out=relu_out, eps=_EPS)
    res = pl.pallas_call(
        kern,
        out_shape=jax.ShapeDtypeStruct((gsz, n * c, lout), jnp.float32),
        grid=(gsz,),
        in_specs=[
            pl.BlockSpec((1, n * c, lin), lambda i: (i, 0, 0)),
            pl.BlockSpec((1, lin, lout), lambda i: (i, 0, 0)),
            pl.BlockSpec((1, 1, lout), lambda i: (i, 0, 0)),
            pl.BlockSpec((1, c, 1), lambda i: (i, 0, 0)),
            pl.BlockSpec((1, c, 1), lambda i: (i, 0, 0)),
        ],
        out_specs=pl.BlockSpec((1, n * c, lout), lambda i: (i, 0, 0)),
        compiler_params=pltpu.CompilerParams(
            dimension_semantics=("parallel",),
            vmem_limit_bytes=_VMEM_LIMIT),
    )(X, W, BIAS, G2, B2)
    return [res[g].reshape(n, c, lout) for g in range(gsz)]


def _convt_bn(xs, wts, gs, bs, *, stride, padding, output_padding,
              relu_in=False, skip=None):
    xds, wcs = [], []
    plo = phi = 0
    for x, wt in zip(xs, wts):
        xd, wc, plo, phi = _dilate_flip(x, wt, stride, padding, output_padding)
        xds.append(xd)
        wcs.append(wc)
    return _conv_bn(xds, wcs, gs, bs, stride=1, pad=(plo, phi),
                    relu_in=relu_in, skip=skip)


def kernel(x, w_enc1, b_enc1, bn1_g, bn1_b, w_enc3, b_enc3, bn3_g, bn3_b,
           w_enc5, b_enc5, bn5_g, bn5_b, W_fc6, b_fc6, bn6_g, bn6_b,
           b1_W_fc, b1_b_fc, b1_bn6x_g, b1_bn6x_b,
           b1_w_dec5, b1_b_dec5, b1_bn5x_g, b1_bn5x_b,
           b1_w_dec3, b1_b_dec3, b1_bn3x_g, b1_bn3x_b,
           b1_w_dec1, b1_b_dec1, b1_bn1x_g, b1_bn1x_b,
           b2_W_fc, b2_b_fc, b2_bn6x_g, b2_bn6x_b,
           b2_w_dec5, b2_b_dec5, b2_bn5x_g, b2_bn5x_b,
           b2_w_dec3, b2_b_dec3, b2_bn3x_g, b2_bn3x_b,
           b2_w_dec1, b2_b_dec1, b2_bn1x_g, b2_bn1x_b):
    (h,) = _conv_bn([x], [w_enc1], [bn1_g], [bn1_b],
                    stride=1, pad=(22, 22), relu_out=True)
    (hs, x0s) = _conv_bn([h], [w_enc3], [bn3_g], [bn3_b],
                         stride=2, pad=(10, 10), relu_out=True, emit_raw=True)
    h, x0 = hs[0], x0s[0]
    (h,) = _conv_bn([h], [w_enc5], [bn5_g], [bn5_b],
                    stride=1, pad=(4, 4), relu_out=True)
    (h,) = _linear_bn([h], [W_fc6], [b_fc6], [bn6_g], [bn6_b], relu_out=True)

    z = _linear_bn([h, h], [b1_W_fc, b2_W_fc], [b1_b_fc, b2_b_fc],
                   [b1_bn6x_g, b2_bn6x_g], [b1_bn6x_b, b2_bn6x_b],
                   relu_in=True)
    u = _convt_bn(z, [b1_w_dec5, b2_w_dec5], [b1_bn5x_g, b2_bn5x_g],
                  [b1_bn5x_b, b2_bn5x_b], stride=1, padding=4,
                  output_padding=0, relu_in=True, skip=x0)
    v = _convt_bn(u, [b1_w_dec3, b2_w_dec3], [b1_bn3x_g, b2_bn3x_g],
                  [b1_bn3x_b, b2_bn3x_b], stride=2, padding=10,
                  output_padding=1, relu_in=True)
    o = _convt_bn(v, [b1_w_dec1, b2_w_dec1], [b1_bn1x_g, b2_bn1x_g],
                  [b1_bn1x_b, b2_bn1x_b], stride=1, padding=22,
                  output_padding=0, relu_in=True)
    return (o[0], o[1])
